# baseline (speedup 1.0000x reference)
import functools

import numpy as np

import jax
import jax.numpy as jnp
from jax import lax
from jax.experimental import pallas as pl
from jax.experimental.pallas import tpu as pltpu
from jax.experimental.pallas import tpu_sc as plsc

N = 10000
NPAD = 10240
E = 320000
NW = 32
EPAD = 327680
EPT = EPAD // NW
CH = 80
NCHUNK = EPT // CH
STRIPE = NPAD // 16
D1 = 128
D2 = 80
BLK = 512


@functools.lru_cache(maxsize=None)
def _gs_kernel(D, tc_tiling=True):
    mesh = plsc.VectorSubcoreMesh(core_axis_name="c", subcore_axis_name="s")

    nbuf = 4
    ring = 2 * nbuf

    @functools.partial(
        pl.kernel,
        out_type=jax.ShapeDtypeStruct((2, NPAD, D), jnp.float32),
        mesh=mesh,
        compiler_params=pltpu.CompilerParams(use_tc_tiling_on_sc=tc_tiling),
        scratch_types=(
            [pltpu.VMEM((ring, CH), jnp.int32),
             pltpu.VMEM((ring, CH), jnp.int32)] +
            [pltpu.VMEM((CH, D), jnp.float32) for _ in range(nbuf)] +
            [pltpu.VMEM_SHARED((NPAD, D), jnp.float32)] +
            [pltpu.SemaphoreType.DMA for _ in range(2 * nbuf + 2)]
        ),
    )
    def gs(g_hbm, ei_hbm, out_hbm, src_v, dst_v, *bufs_and_sems):
        rows = bufs_and_sems[:nbuf]
        acc = bufs_and_sems[nbuf]
        gsem = bufs_and_sems[nbuf + 1:2 * nbuf + 1]
        ssem = bufs_and_sems[2 * nbuf + 1:3 * nbuf + 1]
        isem = bufs_and_sems[3 * nbuf + 1:3 * nbuf + 3]
        c = lax.axis_index("c")
        s = lax.axis_index("s")
        wid = s * 2 + c

        def zrow(i, _):
            for k in range(D // 16):
                rows[0][i, pl.ds(k * 16, 16)] = jnp.zeros((16,), jnp.float32)
            return _
        lax.fori_loop(0, CH, zrow, 0)
        for t in range(STRIPE // CH):
            pltpu.sync_copy(rows[0], acc.at[pl.ds(s * STRIPE + t * CH, CH)])
        plsc.subcore_barrier()

        cbase = wid * NCHUNK

        def _load_idx(j, slot, sem):
            pltpu.async_copy(ei_hbm.at[0, pl.ds(cbase + j, 1)], src_v.at[pl.ds(slot, 1)], sem)
            pltpu.async_copy(ei_hbm.at[1, pl.ds(cbase + j, 1)], dst_v.at[pl.ds(slot, 1)], sem)

        def _wait_idx(slot, sem):
            pltpu.make_async_copy(ei_hbm.at[0, pl.ds(0, 1)], src_v.at[pl.ds(slot, 1)], sem).wait()
            pltpu.make_async_copy(ei_hbm.at[1, pl.ds(0, 1)], dst_v.at[pl.ds(slot, 1)], sem).wait()

        for b in range(nbuf):
            _load_idx(b, b, isem[b % 2])
        for b in range(nbuf):
            _wait_idx(b, isem[b % 2])
        for b in range(nbuf):
            _load_idx(nbuf + b, nbuf + b, isem[b % 2])
        for b in range(nbuf):
            pltpu.async_copy(g_hbm.at[src_v.at[b]], rows[b], gsem[b])

        def body(jj, _):
            j0 = nbuf * jj
            for b in range(nbuf):
                j = j0 + b
                slot = lax.rem(j, ring)
                slot_n = lax.rem(j + nbuf, ring)
                pltpu.make_async_copy(g_hbm.at[src_v.at[slot]], rows[b], gsem[b]).wait()
                pltpu.async_copy(rows[b], acc.at[dst_v.at[slot]], ssem[b], add=True)
                pltpu.make_async_copy(rows[b], acc.at[dst_v.at[slot]], ssem[b]).wait()

                @pl.when(j + nbuf < NCHUNK)
                def _issue():
                    _wait_idx(slot_n, isem[b % 2])
                    pltpu.async_copy(g_hbm.at[src_v.at[slot_n]], rows[b], gsem[b])

                @pl.when(j + 2 * nbuf < NCHUNK)
                def _pre():
                    _load_idx(j + 2 * nbuf, slot, isem[b % 2])
            return _
        lax.fori_loop(0, NCHUNK // nbuf, body, 0)

        plsc.subcore_barrier()
        for t in range(STRIPE // CH):
            r0 = s * STRIPE + t * CH
            pltpu.sync_copy(acc.at[pl.ds(r0, CH)], out_hbm.at[c, pl.ds(r0, CH)])

    return gs


def _deg_call(ei_p):
    mesh = plsc.VectorSubcoreMesh(core_axis_name="c", subcore_axis_name="s")

    @functools.partial(
        pl.kernel,
        out_type=jax.ShapeDtypeStruct((2, NPAD), jnp.float32),
        mesh=mesh,
        scratch_types=[
            pltpu.VMEM((NCHUNK, CH), jnp.int32),
            pltpu.VMEM((CH,), jnp.float32),
            pltpu.VMEM((STRIPE,), jnp.float32),
            pltpu.VMEM_SHARED((NPAD,), jnp.float32),
        ],
    )
    def deg(ei_hbm, out_hbm, idx_v, ones_v, zb_v, acc):
        c = lax.axis_index("c")
        s = lax.axis_index("s")
        wid = s * 2 + c
        for k in range(CH // 16):
            ones_v[pl.ds(k * 16, 16)] = jnp.ones((16,), jnp.float32)
        for k in range(STRIPE // 16):
            zb_v[pl.ds(k * 16, 16)] = jnp.zeros((16,), jnp.float32)
        pltpu.sync_copy(zb_v, acc.at[pl.ds(s * STRIPE, STRIPE)])
        pltpu.sync_copy(ei_hbm.at[1, pl.ds(wid * NCHUNK, NCHUNK)], idx_v)
        plsc.subcore_barrier()

        def body(j, _):
            pltpu.sync_copy(ones_v, acc.at[idx_v.at[j]], add=True)
            return _
        lax.fori_loop(0, NCHUNK, body, 0)

        plsc.subcore_barrier()
        pltpu.sync_copy(acc.at[pl.ds(s * STRIPE, STRIPE)],
                        out_hbm.at[c, pl.ds(s * STRIPE, STRIPE)])

    return deg(ei_p)


def _softplus(x):
    return jnp.maximum(x, 0.0) + jnp.log1p(jnp.exp(-jnp.abs(x)))


def _eye128():
    return (lax.broadcasted_iota(jnp.int32, (128, 128), 0) ==
            lax.broadcasted_iota(jnp.int32, (128, 128), 1)).astype(jnp.float32)


def _to_col(mat):
    r = mat.shape[0]
    dt = lax.dot_general(_eye128(), mat, (((1,), (1,)), ((), ())),
                         preferred_element_type=jnp.float32,
                         precision=lax.Precision.HIGHEST)
    return jnp.concatenate([dt[:, k:k + 1] for k in range(r)], axis=0)


def _to_row(col):
    n = col.shape[0]
    eye = _eye128()
    rows = [lax.dot_general(col[k * 128:(k + 1) * 128, :], eye,
                            (((0,), (0,)), ((), ())),
                            preferred_element_type=jnp.float32,
                            precision=lax.Precision.HIGHEST)
            for k in range(n // 128)]
    return jnp.concatenate(rows, axis=0)


def _dinv_col(degp):
    degp = degp[:, 0]
    d = lax.rsqrt(degp[0] + degp[1] + 1.0)
    return _to_col(d)


def _s1_body(x_ref, w_ref, degp_ref, g_ref):
    dinv = _dinv_col(degp_ref[...])
    g_ref[...] = jnp.dot(x_ref[...], w_ref[...],
                         preferred_element_type=jnp.float32) * dinv


def _l2_body(acc_ref, g1_ref, degp_ref, w_ref, g2_ref):
    dinv = _dinv_col(degp_ref[...])
    a = acc_ref[...]
    out1 = _softplus((a[0] + a[1] + g1_ref[...]) * dinv)
    g2_ref[...] = jnp.dot(out1, w_ref[...],
                          preferred_element_type=jnp.float32) * dinv


def _gamma(z):
    a = jnp.float32(0.99999999999980993)
    for i, ci in enumerate([
            676.5203681218851, -1259.1392167224028, 771.32342877765313,
            -176.61502916214059, 12.507343278686905, -0.13857109526572012,
            9.9843695780195716e-6, 1.5056327351493116e-7]):
        a = a + jnp.float32(ci) / (z + jnp.float32(i))
    t = z + 6.5
    return jnp.float32(2.5066282746310002) * jnp.exp(
        (z - 0.5) * jnp.log(t) - t) * a


def _fin_body(acc_ref, g2_ref, degp_ref, z_ref, lbd_ref, kap_ref):
    dinv = _dinv_col(degp_ref[...])
    a = acc_ref[...]
    h = _softplus((a[0] + a[1] + g2_ref[...]) * dinv)
    lbd = h[:, :64]
    kap = h[:, 64:65] + 0.1
    lbd_ref[...] = lbd
    kap_ref[...] = kap
    kap_row = _to_row(kap)
    gam = _to_col(_gamma(1.0 + 1.0 / kap_row))
    z_ref[...] = lbd * gam


_PAD2D = np.broadcast_to(
    np.asarray(N + (np.arange(EPAD - E) % (NPAD - N)), dtype=np.int32),
    (2, EPAD - E))


def kernel(x, edge_index, W1, W2, mask_rate):
    del mask_rate
    f32 = jnp.float32
    npad_extra = NPAD - N
    ei_p = jnp.concatenate(
        [edge_index, jnp.asarray(_PAD2D)], axis=1).reshape(2, NW * NCHUNK, CH)
    x_p = jnp.concatenate([x, jnp.zeros((npad_extra, x.shape[1]), f32)])
    w2_p = jnp.pad(W2, ((0, 0), (0, D2 - W2.shape[1])))

    nblk = NPAD // BLK
    r = BLK // 128
    degp = _deg_call(ei_p).reshape(2, nblk, r, 128)

    g1 = pl.pallas_call(
        _s1_body,
        grid=(nblk,),
        in_specs=[
            pl.BlockSpec((BLK, D1), lambda i: (i, 0)),
            pl.BlockSpec((D1, D1), lambda i: (0, 0)),
            pl.BlockSpec((2, 1, r, 128), lambda i: (0, i, 0, 0)),
        ],
        out_specs=pl.BlockSpec((BLK, D1), lambda i: (i, 0)),
        out_shape=jax.ShapeDtypeStruct((NPAD, D1), f32),
    )(x_p, W1, degp)

    acc1 = _gs_kernel(D1)(g1, ei_p)

    g2 = pl.pallas_call(
        _l2_body,
        grid=(nblk,),
        in_specs=[
            pl.BlockSpec((2, BLK, D1), lambda i: (0, i, 0)),
            pl.BlockSpec((BLK, D1), lambda i: (i, 0)),
            pl.BlockSpec((2, 1, r, 128), lambda i: (0, i, 0, 0)),
            pl.BlockSpec((D1, D2), lambda i: (0, 0)),
        ],
        out_specs=pl.BlockSpec((BLK, D2), lambda i: (i, 0)),
        out_shape=jax.ShapeDtypeStruct((NPAD, D2), f32),
    )(acc1, g1, degp, w2_p)

    acc2 = _gs_kernel(D2, tc_tiling=False)(g2, ei_p)

    z, lbd, kap = pl.pallas_call(
        _fin_body,
        grid=(nblk,),
        in_specs=[
            pl.BlockSpec((2, BLK, D2), lambda i: (0, i, 0)),
            pl.BlockSpec((BLK, D2), lambda i: (i, 0)),
            pl.BlockSpec((2, 1, r, 128), lambda i: (0, i, 0, 0)),
        ],
        out_specs=[
            pl.BlockSpec((BLK, 64), lambda i: (i, 0)),
            pl.BlockSpec((BLK, 64), lambda i: (i, 0)),
            pl.BlockSpec((BLK, 1), lambda i: (i, 0)),
        ],
        out_shape=[
            jax.ShapeDtypeStruct((N, 64), f32),
            jax.ShapeDtypeStruct((N, 64), f32),
            jax.ShapeDtypeStruct((N, 1), f32),
        ],
    )(acc2, g2, degp)

    return (z, lbd, kap)


# speedup vs baseline: 1.0838x; 1.0838x over previous
import functools

import numpy as np

import jax
import jax.numpy as jnp
from jax import lax
from jax.experimental import pallas as pl
from jax.experimental.pallas import tpu as pltpu
from jax.experimental.pallas import tpu_sc as plsc

N = 10000
NPAD = 10240
E = 320000
NW = 32
EPAD = 327680
EPT = EPAD // NW
CH = 80
NCHUNK = EPT // CH
STRIPE = NPAD // 16
D1 = 128
D2 = 80
BLK = 1024


@functools.lru_cache(maxsize=None)
def _gs_kernel(D, tc_tiling=True):
    mesh = plsc.VectorSubcoreMesh(core_axis_name="c", subcore_axis_name="s")

    nbuf = 4
    ring = 2 * nbuf

    @functools.partial(
        pl.kernel,
        out_type=jax.ShapeDtypeStruct((2, NPAD, D), jnp.float32),
        mesh=mesh,
        compiler_params=pltpu.CompilerParams(use_tc_tiling_on_sc=tc_tiling),
        scratch_types=(
            [pltpu.VMEM((ring, CH), jnp.int32),
             pltpu.VMEM((ring, CH), jnp.int32)] +
            [pltpu.VMEM((CH, D), jnp.float32) for _ in range(nbuf)] +
            [pltpu.VMEM_SHARED((NPAD, D), jnp.float32)] +
            [pltpu.SemaphoreType.DMA for _ in range(2 * nbuf + 2)]
        ),
    )
    def gs(g_hbm, ei_hbm, out_hbm, src_v, dst_v, *bufs_and_sems):
        rows = bufs_and_sems[:nbuf]
        acc = bufs_and_sems[nbuf]
        gsem = bufs_and_sems[nbuf + 1:2 * nbuf + 1]
        ssem = bufs_and_sems[2 * nbuf + 1:3 * nbuf + 1]
        isem = bufs_and_sems[3 * nbuf + 1:3 * nbuf + 3]
        c = lax.axis_index("c")
        s = lax.axis_index("s")
        wid = s * 2 + c

        def zrow(i, _):
            for k in range(D // 16):
                rows[0][i, pl.ds(k * 16, 16)] = jnp.zeros((16,), jnp.float32)
            return _
        lax.fori_loop(0, CH, zrow, 0)
        for t in range(STRIPE // CH):
            pltpu.sync_copy(rows[0], acc.at[pl.ds(s * STRIPE + t * CH, CH)])
        plsc.subcore_barrier()

        cbase = wid * NCHUNK

        def _load_idx(j, slot, sem):
            pltpu.async_copy(ei_hbm.at[0, pl.ds(cbase + j, 1)], src_v.at[pl.ds(slot, 1)], sem)
            pltpu.async_copy(ei_hbm.at[1, pl.ds(cbase + j, 1)], dst_v.at[pl.ds(slot, 1)], sem)

        def _wait_idx(slot, sem):
            pltpu.make_async_copy(ei_hbm.at[0, pl.ds(0, 1)], src_v.at[pl.ds(slot, 1)], sem).wait()
            pltpu.make_async_copy(ei_hbm.at[1, pl.ds(0, 1)], dst_v.at[pl.ds(slot, 1)], sem).wait()

        for b in range(nbuf):
            _load_idx(b, b, isem[b % 2])
        for b in range(nbuf):
            _wait_idx(b, isem[b % 2])
        for b in range(nbuf):
            _load_idx(nbuf + b, nbuf + b, isem[b % 2])
        for b in range(nbuf):
            pltpu.async_copy(g_hbm.at[src_v.at[b]], rows[b], gsem[b])

        def body(jj, _):
            j0 = nbuf * jj
            for b in range(nbuf):
                j = j0 + b
                slot = lax.rem(j, ring)
                slot_n = lax.rem(j + nbuf, ring)
                pltpu.make_async_copy(g_hbm.at[src_v.at[slot]], rows[b], gsem[b]).wait()
                pltpu.async_copy(rows[b], acc.at[dst_v.at[slot]], ssem[b], add=True)
                pltpu.make_async_copy(rows[b], acc.at[dst_v.at[slot]], ssem[b]).wait()

                @pl.when(j + nbuf < NCHUNK)
                def _issue():
                    _wait_idx(slot_n, isem[b % 2])
                    pltpu.async_copy(g_hbm.at[src_v.at[slot_n]], rows[b], gsem[b])

                @pl.when(j + 2 * nbuf < NCHUNK)
                def _pre():
                    _load_idx(j + 2 * nbuf, slot, isem[b % 2])
            return _
        lax.fori_loop(0, NCHUNK // nbuf, body, 0)

        plsc.subcore_barrier()
        for t in range(STRIPE // CH):
            r0 = s * STRIPE + t * CH
            pltpu.sync_copy(acc.at[pl.ds(r0, CH)], out_hbm.at[c, pl.ds(r0, CH)])

    return gs


def _deg_call(ei_p):
    mesh = plsc.VectorSubcoreMesh(core_axis_name="c", subcore_axis_name="s")

    @functools.partial(
        pl.kernel,
        out_type=jax.ShapeDtypeStruct((2, NPAD), jnp.float32),
        mesh=mesh,
        scratch_types=[
            pltpu.VMEM((NCHUNK, CH), jnp.int32),
            pltpu.VMEM((CH,), jnp.float32),
            pltpu.VMEM((STRIPE,), jnp.float32),
            pltpu.VMEM_SHARED((NPAD,), jnp.float32),
        ],
    )
    def deg(ei_hbm, out_hbm, idx_v, ones_v, zb_v, acc):
        c = lax.axis_index("c")
        s = lax.axis_index("s")
        wid = s * 2 + c
        for k in range(CH // 16):
            ones_v[pl.ds(k * 16, 16)] = jnp.ones((16,), jnp.float32)
        for k in range(STRIPE // 16):
            zb_v[pl.ds(k * 16, 16)] = jnp.zeros((16,), jnp.float32)
        pltpu.sync_copy(zb_v, acc.at[pl.ds(s * STRIPE, STRIPE)])
        pltpu.sync_copy(ei_hbm.at[1, pl.ds(wid * NCHUNK, NCHUNK)], idx_v)
        plsc.subcore_barrier()

        def body(j, _):
            pltpu.sync_copy(ones_v, acc.at[idx_v.at[j]], add=True)
            return _
        lax.fori_loop(0, NCHUNK, body, 0)

        plsc.subcore_barrier()
        pltpu.sync_copy(acc.at[pl.ds(s * STRIPE, STRIPE)],
                        out_hbm.at[c, pl.ds(s * STRIPE, STRIPE)])

    return deg(ei_p)


def _softplus(x):
    return jnp.maximum(x, 0.0) + jnp.log1p(jnp.exp(-jnp.abs(x)))


def _eye128():
    return (lax.broadcasted_iota(jnp.int32, (128, 128), 0) ==
            lax.broadcasted_iota(jnp.int32, (128, 128), 1)).astype(jnp.float32)


def _to_col(mat):
    r = mat.shape[0]
    dt = lax.dot_general(_eye128(), mat, (((1,), (1,)), ((), ())),
                         preferred_element_type=jnp.float32,
                         precision=lax.Precision.HIGHEST)
    return jnp.concatenate([dt[:, k:k + 1] for k in range(r)], axis=0)


def _dinv_col(degp):
    degp = degp[:, 0]
    d = lax.rsqrt(degp[0] + degp[1] + 1.0)
    return _to_col(d)


def _s1_body(x_ref, w_ref, degp_ref, g_ref):
    dinv = _dinv_col(degp_ref[...])
    g_ref[...] = jnp.dot(x_ref[...], w_ref[...],
                         preferred_element_type=jnp.float32) * dinv


def _l2_body(acc_ref, g1_ref, degp_ref, w_ref, g2_ref):
    dinv = _dinv_col(degp_ref[...])
    a = acc_ref[...]
    out1 = _softplus((a[0] + a[1] + g1_ref[...]) * dinv)
    g2_ref[...] = jnp.dot(out1, w_ref[...],
                          preferred_element_type=jnp.float32) * dinv


def _gamma(z):
    a = jnp.float32(0.99999999999980993)
    for i, ci in enumerate([
            676.5203681218851, -1259.1392167224028, 771.32342877765313,
            -176.61502916214059, 12.507343278686905, -0.13857109526572012,
            9.9843695780195716e-6, 1.5056327351493116e-7]):
        a = a + jnp.float32(ci) / (z + jnp.float32(i))
    t = z + 6.5
    return jnp.float32(2.5066282746310002) * jnp.exp(
        (z - 0.5) * jnp.log(t) - t) * a


def _fin_body(acc_ref, g2_ref, degp_ref, z_ref, lbd_ref, kap_ref):
    dinv = _dinv_col(degp_ref[...])
    a = acc_ref[...]
    h = _softplus((a[0] + a[1] + g2_ref[...]) * dinv)
    lbd = h[:, :64]
    kap = h[:, 64:65] + 0.1
    lbd_ref[...] = lbd
    kap_ref[...] = kap
    z_ref[...] = lbd * _gamma(1.0 + 1.0 / kap)


_PAD2D = np.broadcast_to(
    np.asarray(N + (np.arange(EPAD - E) % (NPAD - N)), dtype=np.int32),
    (2, EPAD - E))


def kernel(x, edge_index, W1, W2, mask_rate):
    del mask_rate
    f32 = jnp.float32
    npad_extra = NPAD - N
    ei_p = jnp.concatenate(
        [edge_index, jnp.asarray(_PAD2D)], axis=1).reshape(2, NW * NCHUNK, CH)
    x_p = jnp.concatenate([x, jnp.zeros((npad_extra, x.shape[1]), f32)])
    w2_p = jnp.pad(W2, ((0, 0), (0, D2 - W2.shape[1])))

    nblk = NPAD // BLK
    r = BLK // 128
    degp = _deg_call(ei_p).reshape(2, nblk, r, 128)

    g1 = pl.pallas_call(
        _s1_body,
        grid=(nblk,),
        in_specs=[
            pl.BlockSpec((BLK, D1), lambda i: (i, 0)),
            pl.BlockSpec((D1, D1), lambda i: (0, 0)),
            pl.BlockSpec((2, 1, r, 128), lambda i: (0, i, 0, 0)),
        ],
        out_specs=pl.BlockSpec((BLK, D1), lambda i: (i, 0)),
        out_shape=jax.ShapeDtypeStruct((NPAD, D1), f32),
    )(x_p, W1, degp)

    acc1 = _gs_kernel(D1)(g1, ei_p)

    g2 = pl.pallas_call(
        _l2_body,
        grid=(nblk,),
        in_specs=[
            pl.BlockSpec((2, BLK, D1), lambda i: (0, i, 0)),
            pl.BlockSpec((BLK, D1), lambda i: (i, 0)),
            pl.BlockSpec((2, 1, r, 128), lambda i: (0, i, 0, 0)),
            pl.BlockSpec((D1, D2), lambda i: (0, 0)),
        ],
        out_specs=pl.BlockSpec((BLK, D2), lambda i: (i, 0)),
        out_shape=jax.ShapeDtypeStruct((NPAD, D2), f32),
    )(acc1, g1, degp, w2_p)

    acc2 = _gs_kernel(D2, tc_tiling=False)(g2, ei_p)

    z, lbd, kap = pl.pallas_call(
        _fin_body,
        grid=(nblk,),
        in_specs=[
            pl.BlockSpec((2, BLK, D2), lambda i: (0, i, 0)),
            pl.BlockSpec((BLK, D2), lambda i: (i, 0)),
            pl.BlockSpec((2, 1, r, 128), lambda i: (0, i, 0, 0)),
        ],
        out_specs=[
            pl.BlockSpec((BLK, 64), lambda i: (i, 0)),
            pl.BlockSpec((BLK, 64), lambda i: (i, 0)),
            pl.BlockSpec((BLK, 1), lambda i: (i, 0)),
        ],
        out_shape=[
            jax.ShapeDtypeStruct((N, 64), f32),
            jax.ShapeDtypeStruct((N, 64), f32),
            jax.ShapeDtypeStruct((N, 1), f32),
        ],
    )(acc2, g2, degp)

    return (z, lbd, kap)


# speedup vs baseline: 1.1120x; 1.0260x over previous
import functools

import numpy as np

import jax
import jax.numpy as jnp
from jax import lax
from jax.experimental import pallas as pl
from jax.experimental.pallas import tpu as pltpu
from jax.experimental.pallas import tpu_sc as plsc

N = 10000
NPAD = 10240
E = 320000
NW = 32
EPAD = 327680
EPT = EPAD // NW
CH = 80
NCHUNK = EPT // CH
STRIPE = NPAD // 16
D1 = 128
D2 = 80
BLK = 2048


@functools.lru_cache(maxsize=None)
def _gs_kernel(D, tc_tiling=True):
    mesh = plsc.VectorSubcoreMesh(core_axis_name="c", subcore_axis_name="s")

    nbuf = 4
    ring = 2 * nbuf

    @functools.partial(
        pl.kernel,
        out_type=jax.ShapeDtypeStruct((2, NPAD, D), jnp.float32),
        mesh=mesh,
        compiler_params=pltpu.CompilerParams(use_tc_tiling_on_sc=tc_tiling),
        scratch_types=(
            [pltpu.VMEM((ring, CH), jnp.int32),
             pltpu.VMEM((ring, CH), jnp.int32)] +
            [pltpu.VMEM((CH, D), jnp.float32) for _ in range(nbuf)] +
            [pltpu.VMEM_SHARED((NPAD, D), jnp.float32)] +
            [pltpu.SemaphoreType.DMA for _ in range(2 * nbuf + 2)]
        ),
    )
    def gs(g_hbm, ei_hbm, out_hbm, src_v, dst_v, *bufs_and_sems):
        rows = bufs_and_sems[:nbuf]
        acc = bufs_and_sems[nbuf]
        gsem = bufs_and_sems[nbuf + 1:2 * nbuf + 1]
        ssem = bufs_and_sems[2 * nbuf + 1:3 * nbuf + 1]
        isem = bufs_and_sems[3 * nbuf + 1:3 * nbuf + 3]
        c = lax.axis_index("c")
        s = lax.axis_index("s")
        wid = s * 2 + c

        def zrow(i, _):
            for k in range(D // 16):
                rows[0][i, pl.ds(k * 16, 16)] = jnp.zeros((16,), jnp.float32)
            return _
        lax.fori_loop(0, CH, zrow, 0)
        for t in range(STRIPE // CH):
            pltpu.sync_copy(rows[0], acc.at[pl.ds(s * STRIPE + t * CH, CH)])
        plsc.subcore_barrier()

        cbase = wid * NCHUNK

        def _load_idx(j, slot, sem):
            pltpu.async_copy(ei_hbm.at[0, pl.ds(cbase + j, 1)], src_v.at[pl.ds(slot, 1)], sem)
            pltpu.async_copy(ei_hbm.at[1, pl.ds(cbase + j, 1)], dst_v.at[pl.ds(slot, 1)], sem)

        def _wait_idx(slot, sem):
            pltpu.make_async_copy(ei_hbm.at[0, pl.ds(0, 1)], src_v.at[pl.ds(slot, 1)], sem).wait()
            pltpu.make_async_copy(ei_hbm.at[1, pl.ds(0, 1)], dst_v.at[pl.ds(slot, 1)], sem).wait()

        for b in range(nbuf):
            _load_idx(b, b, isem[b % 2])
        for b in range(nbuf):
            _wait_idx(b, isem[b % 2])
        for b in range(nbuf):
            _load_idx(nbuf + b, nbuf + b, isem[b % 2])
        for b in range(nbuf):
            pltpu.async_copy(g_hbm.at[src_v.at[b]], rows[b], gsem[b])

        def body(jj, _):
            j0 = nbuf * jj
            for b in range(nbuf):
                j = j0 + b
                slot = lax.rem(j, ring)
                slot_n = lax.rem(j + nbuf, ring)
                pltpu.make_async_copy(g_hbm.at[src_v.at[slot]], rows[b], gsem[b]).wait()
                pltpu.async_copy(rows[b], acc.at[dst_v.at[slot]], ssem[b], add=True)
                pltpu.make_async_copy(rows[b], acc.at[dst_v.at[slot]], ssem[b]).wait()

                @pl.when(j + nbuf < NCHUNK)
                def _issue():
                    _wait_idx(slot_n, isem[b % 2])
                    pltpu.async_copy(g_hbm.at[src_v.at[slot_n]], rows[b], gsem[b])

                @pl.when(j + 2 * nbuf < NCHUNK)
                def _pre():
                    _load_idx(j + 2 * nbuf, slot, isem[b % 2])
            return _
        lax.fori_loop(0, NCHUNK // nbuf, body, 0)

        plsc.subcore_barrier()
        for t in range(STRIPE // CH):
            r0 = s * STRIPE + t * CH
            pltpu.sync_copy(acc.at[pl.ds(r0, CH)], out_hbm.at[c, pl.ds(r0, CH)])

    return gs


def _deg_call(ei_p):
    mesh = plsc.VectorSubcoreMesh(core_axis_name="c", subcore_axis_name="s")

    @functools.partial(
        pl.kernel,
        out_type=jax.ShapeDtypeStruct((2, NPAD), jnp.float32),
        mesh=mesh,
        scratch_types=[
            pltpu.VMEM((NCHUNK, CH), jnp.int32),
            pltpu.VMEM((CH,), jnp.float32),
            pltpu.VMEM((STRIPE,), jnp.float32),
            pltpu.VMEM_SHARED((NPAD,), jnp.float32),
        ],
    )
    def deg(ei_hbm, out_hbm, idx_v, ones_v, zb_v, acc):
        c = lax.axis_index("c")
        s = lax.axis_index("s")
        wid = s * 2 + c
        for k in range(CH // 16):
            ones_v[pl.ds(k * 16, 16)] = jnp.ones((16,), jnp.float32)
        for k in range(STRIPE // 16):
            zb_v[pl.ds(k * 16, 16)] = jnp.zeros((16,), jnp.float32)
        pltpu.sync_copy(zb_v, acc.at[pl.ds(s * STRIPE, STRIPE)])
        pltpu.sync_copy(ei_hbm.at[1, pl.ds(wid * NCHUNK, NCHUNK)], idx_v)
        plsc.subcore_barrier()

        def body(j, _):
            pltpu.sync_copy(ones_v, acc.at[idx_v.at[j]], add=True)
            return _
        lax.fori_loop(0, NCHUNK, body, 0)

        plsc.subcore_barrier()
        pltpu.sync_copy(acc.at[pl.ds(s * STRIPE, STRIPE)],
                        out_hbm.at[c, pl.ds(s * STRIPE, STRIPE)])

    return deg(ei_p)


def _softplus(x):
    return jnp.maximum(x, 0.0) + jnp.log1p(jnp.exp(-jnp.abs(x)))


def _eye128():
    return (lax.broadcasted_iota(jnp.int32, (128, 128), 0) ==
            lax.broadcasted_iota(jnp.int32, (128, 128), 1)).astype(jnp.float32)


def _to_col(mat):
    r = mat.shape[0]
    dt = lax.dot_general(_eye128(), mat, (((1,), (1,)), ((), ())),
                         preferred_element_type=jnp.float32,
                         precision=lax.Precision.HIGHEST)
    return jnp.concatenate([dt[:, k:k + 1] for k in range(r)], axis=0)


def _dinv_col(degp):
    degp = degp[:, 0]
    d = lax.rsqrt(degp[0] + degp[1] + 1.0)
    return _to_col(d)


def _s1_body(x_ref, w_ref, degp_ref, g_ref):
    dinv = _dinv_col(degp_ref[...])
    g_ref[...] = jnp.dot(x_ref[...], w_ref[...],
                         preferred_element_type=jnp.float32) * dinv


def _l2_body(acc_ref, g1_ref, degp_ref, w_ref, g2_ref):
    dinv = _dinv_col(degp_ref[...])
    a = acc_ref[...]
    out1 = _softplus((a[0] + a[1] + g1_ref[...]) * dinv)
    g2_ref[...] = jnp.dot(out1, w_ref[...],
                          preferred_element_type=jnp.float32) * dinv


def _gamma(z):
    a = jnp.float32(0.99999999999980993)
    for i, ci in enumerate([
            676.5203681218851, -1259.1392167224028, 771.32342877765313,
            -176.61502916214059, 12.507343278686905, -0.13857109526572012,
            9.9843695780195716e-6, 1.5056327351493116e-7]):
        a = a + jnp.float32(ci) / (z + jnp.float32(i))
    t = z + 6.5
    return jnp.float32(2.5066282746310002) * jnp.exp(
        (z - 0.5) * jnp.log(t) - t) * a


def _fin_body(acc_ref, g2_ref, degp_ref, z_ref, lbd_ref, kap_ref):
    dinv = _dinv_col(degp_ref[...])
    a = acc_ref[...]
    h = _softplus((a[0] + a[1] + g2_ref[...]) * dinv)
    lbd = h[:, :64]
    kap = h[:, 64:65] + 0.1
    lbd_ref[...] = lbd
    kap_ref[...] = kap
    z_ref[...] = lbd * _gamma(1.0 + 1.0 / kap)


_PAD2D = np.broadcast_to(
    np.asarray(N + (np.arange(EPAD - E) % (NPAD - N)), dtype=np.int32),
    (2, EPAD - E))


def kernel(x, edge_index, W1, W2, mask_rate):
    del mask_rate
    f32 = jnp.float32
    npad_extra = NPAD - N
    ei_p = jnp.concatenate(
        [edge_index, jnp.asarray(_PAD2D)], axis=1).reshape(2, NW * NCHUNK, CH)
    x_p = jnp.concatenate([x, jnp.zeros((npad_extra, x.shape[1]), f32)])
    w2_p = jnp.pad(W2, ((0, 0), (0, D2 - W2.shape[1])))

    nblk = NPAD // BLK
    r = BLK // 128
    degp = _deg_call(ei_p).reshape(2, nblk, r, 128)

    g1 = pl.pallas_call(
        _s1_body,
        grid=(nblk,),
        in_specs=[
            pl.BlockSpec((BLK, D1), lambda i: (i, 0)),
            pl.BlockSpec((D1, D1), lambda i: (0, 0)),
            pl.BlockSpec((2, 1, r, 128), lambda i: (0, i, 0, 0)),
        ],
        out_specs=pl.BlockSpec((BLK, D1), lambda i: (i, 0)),
        out_shape=jax.ShapeDtypeStruct((NPAD, D1), f32),
    )(x_p, W1, degp)

    acc1 = _gs_kernel(D1)(g1, ei_p)

    g2 = pl.pallas_call(
        _l2_body,
        grid=(nblk,),
        in_specs=[
            pl.BlockSpec((2, BLK, D1), lambda i: (0, i, 0)),
            pl.BlockSpec((BLK, D1), lambda i: (i, 0)),
            pl.BlockSpec((2, 1, r, 128), lambda i: (0, i, 0, 0)),
            pl.BlockSpec((D1, D2), lambda i: (0, 0)),
        ],
        out_specs=pl.BlockSpec((BLK, D2), lambda i: (i, 0)),
        out_shape=jax.ShapeDtypeStruct((NPAD, D2), f32),
    )(acc1, g1, degp, w2_p)

    acc2 = _gs_kernel(D2, tc_tiling=False)(g2, ei_p)

    z, lbd, kap = pl.pallas_call(
        _fin_body,
        grid=(nblk,),
        in_specs=[
            pl.BlockSpec((2, BLK, D2), lambda i: (0, i, 0)),
            pl.BlockSpec((BLK, D2), lambda i: (i, 0)),
            pl.BlockSpec((2, 1, r, 128), lambda i: (0, i, 0, 0)),
        ],
        out_specs=[
            pl.BlockSpec((BLK, 64), lambda i: (i, 0)),
            pl.BlockSpec((BLK, 64), lambda i: (i, 0)),
            pl.BlockSpec((BLK, 1), lambda i: (i, 0)),
        ],
        out_shape=[
            jax.ShapeDtypeStruct((N, 64), f32),
            jax.ShapeDtypeStruct((N, 64), f32),
            jax.ShapeDtypeStruct((N, 1), f32),
        ],
    )(acc2, g2, degp)

    return (z, lbd, kap)


# speedup vs baseline: 1.1185x; 1.0058x over previous
import functools

import numpy as np

import jax
import jax.numpy as jnp
from jax import lax
from jax.experimental import pallas as pl
from jax.experimental.pallas import tpu as pltpu
from jax.experimental.pallas import tpu_sc as plsc

N = 10000
NPAD = 10240
E = 320000
NW = 32
EPAD = 327680
EPT = EPAD // NW
CH = 80
NCHUNK = EPT // CH
STRIPE = NPAD // 16
D1 = 128
D2 = 80
BLK = 2560


@functools.lru_cache(maxsize=None)
def _gs_kernel(D, tc_tiling=True):
    mesh = plsc.VectorSubcoreMesh(core_axis_name="c", subcore_axis_name="s")

    nbuf = 4
    ring = 2 * nbuf

    @functools.partial(
        pl.kernel,
        out_type=jax.ShapeDtypeStruct((2, NPAD, D), jnp.float32),
        mesh=mesh,
        compiler_params=pltpu.CompilerParams(use_tc_tiling_on_sc=tc_tiling),
        scratch_types=(
            [pltpu.VMEM((ring, CH), jnp.int32),
             pltpu.VMEM((ring, CH), jnp.int32)] +
            [pltpu.VMEM((CH, D), jnp.float32) for _ in range(nbuf)] +
            [pltpu.VMEM_SHARED((NPAD, D), jnp.float32)] +
            [pltpu.SemaphoreType.DMA for _ in range(2 * nbuf + 2)]
        ),
    )
    def gs(g_hbm, ei_hbm, out_hbm, src_v, dst_v, *bufs_and_sems):
        rows = bufs_and_sems[:nbuf]
        acc = bufs_and_sems[nbuf]
        gsem = bufs_and_sems[nbuf + 1:2 * nbuf + 1]
        ssem = bufs_and_sems[2 * nbuf + 1:3 * nbuf + 1]
        isem = bufs_and_sems[3 * nbuf + 1:3 * nbuf + 3]
        c = lax.axis_index("c")
        s = lax.axis_index("s")
        wid = s * 2 + c

        def zrow(i, _):
            for k in range(D // 16):
                rows[0][i, pl.ds(k * 16, 16)] = jnp.zeros((16,), jnp.float32)
            return _
        lax.fori_loop(0, CH, zrow, 0)
        for t in range(STRIPE // CH):
            pltpu.sync_copy(rows[0], acc.at[pl.ds(s * STRIPE + t * CH, CH)])
        plsc.subcore_barrier()

        cbase = wid * NCHUNK

        def _load_idx(j, slot, sem):
            pltpu.async_copy(ei_hbm.at[0, pl.ds(cbase + j, 1)], src_v.at[pl.ds(slot, 1)], sem)
            pltpu.async_copy(ei_hbm.at[1, pl.ds(cbase + j, 1)], dst_v.at[pl.ds(slot, 1)], sem)

        def _wait_idx(slot, sem):
            pltpu.make_async_copy(ei_hbm.at[0, pl.ds(0, 1)], src_v.at[pl.ds(slot, 1)], sem).wait()
            pltpu.make_async_copy(ei_hbm.at[1, pl.ds(0, 1)], dst_v.at[pl.ds(slot, 1)], sem).wait()

        for b in range(nbuf):
            _load_idx(b, b, isem[b % 2])
        for b in range(nbuf):
            _wait_idx(b, isem[b % 2])
        for b in range(nbuf):
            _load_idx(nbuf + b, nbuf + b, isem[b % 2])
        for b in range(nbuf):
            pltpu.async_copy(g_hbm.at[src_v.at[b]], rows[b], gsem[b])

        def body(jj, _):
            j0 = nbuf * jj
            for b in range(nbuf):
                j = j0 + b
                slot = lax.rem(j, ring)
                slot_n = lax.rem(j + nbuf, ring)
                pltpu.make_async_copy(g_hbm.at[src_v.at[slot]], rows[b], gsem[b]).wait()
                pltpu.async_copy(rows[b], acc.at[dst_v.at[slot]], ssem[b], add=True)
                pltpu.make_async_copy(rows[b], acc.at[dst_v.at[slot]], ssem[b]).wait()

                @pl.when(j + nbuf < NCHUNK)
                def _issue():
                    _wait_idx(slot_n, isem[b % 2])
                    pltpu.async_copy(g_hbm.at[src_v.at[slot_n]], rows[b], gsem[b])

                @pl.when(j + 2 * nbuf < NCHUNK)
                def _pre():
                    _load_idx(j + 2 * nbuf, slot, isem[b % 2])
            return _
        lax.fori_loop(0, NCHUNK // nbuf, body, 0)

        plsc.subcore_barrier()
        for t in range(STRIPE // CH):
            r0 = s * STRIPE + t * CH
            pltpu.sync_copy(acc.at[pl.ds(r0, CH)], out_hbm.at[c, pl.ds(r0, CH)])

    return gs


def _deg_call(ei_p):
    mesh = plsc.VectorSubcoreMesh(core_axis_name="c", subcore_axis_name="s")

    @functools.partial(
        pl.kernel,
        out_type=jax.ShapeDtypeStruct((2, NPAD), jnp.float32),
        mesh=mesh,
        scratch_types=[
            pltpu.VMEM((NCHUNK, CH), jnp.int32),
            pltpu.VMEM((CH,), jnp.float32),
            pltpu.VMEM((STRIPE,), jnp.float32),
            pltpu.VMEM_SHARED((NPAD,), jnp.float32),
        ],
    )
    def deg(ei_hbm, out_hbm, idx_v, ones_v, zb_v, acc):
        c = lax.axis_index("c")
        s = lax.axis_index("s")
        wid = s * 2 + c
        for k in range(CH // 16):
            ones_v[pl.ds(k * 16, 16)] = jnp.ones((16,), jnp.float32)
        for k in range(STRIPE // 16):
            zb_v[pl.ds(k * 16, 16)] = jnp.zeros((16,), jnp.float32)
        pltpu.sync_copy(zb_v, acc.at[pl.ds(s * STRIPE, STRIPE)])
        pltpu.sync_copy(ei_hbm.at[1, pl.ds(wid * NCHUNK, NCHUNK)], idx_v)
        plsc.subcore_barrier()

        def body(j, _):
            pltpu.sync_copy(ones_v, acc.at[idx_v.at[j]], add=True)
            return _
        lax.fori_loop(0, NCHUNK, body, 0)

        plsc.subcore_barrier()
        pltpu.sync_copy(acc.at[pl.ds(s * STRIPE, STRIPE)],
                        out_hbm.at[c, pl.ds(s * STRIPE, STRIPE)])

    return deg(ei_p)


def _softplus(x):
    return jnp.maximum(x, 0.0) + jnp.log1p(jnp.exp(-jnp.abs(x)))


def _eye128():
    return (lax.broadcasted_iota(jnp.int32, (128, 128), 0) ==
            lax.broadcasted_iota(jnp.int32, (128, 128), 1)).astype(jnp.float32)


def _to_col(mat):
    r = mat.shape[0]
    dt = lax.dot_general(_eye128(), mat, (((1,), (1,)), ((), ())),
                         preferred_element_type=jnp.float32,
                         precision=lax.Precision.HIGHEST)
    return jnp.concatenate([dt[:, k:k + 1] for k in range(r)], axis=0)


def _dinv_col(degp):
    degp = degp[:, 0]
    d = lax.rsqrt(degp[0] + degp[1] + 1.0)
    return _to_col(d)


def _s1_body(x_ref, w_ref, degp_ref, g_ref):
    dinv = _dinv_col(degp_ref[...])
    g_ref[...] = jnp.dot(x_ref[...], w_ref[...],
                         preferred_element_type=jnp.float32) * dinv


def _l2_body(acc_ref, g1_ref, degp_ref, w_ref, g2_ref):
    dinv = _dinv_col(degp_ref[...])
    a = acc_ref[...]
    out1 = _softplus((a[0] + a[1] + g1_ref[...]) * dinv)
    g2_ref[...] = jnp.dot(out1, w_ref[...],
                          preferred_element_type=jnp.float32) * dinv


def _gamma(z):
    a = jnp.float32(0.99999999999980993)
    for i, ci in enumerate([
            676.5203681218851, -1259.1392167224028, 771.32342877765313,
            -176.61502916214059, 12.507343278686905, -0.13857109526572012,
            9.9843695780195716e-6, 1.5056327351493116e-7]):
        a = a + jnp.float32(ci) / (z + jnp.float32(i))
    t = z + 6.5
    return jnp.float32(2.5066282746310002) * jnp.exp(
        (z - 0.5) * jnp.log(t) - t) * a


def _fin_body(acc_ref, g2_ref, degp_ref, z_ref, lbd_ref, kap_ref):
    dinv = _dinv_col(degp_ref[...])
    a = acc_ref[...]
    h = _softplus((a[0] + a[1] + g2_ref[...]) * dinv)
    lbd = h[:, :64]
    kap = h[:, 64:65] + 0.1
    lbd_ref[...] = lbd
    kap_ref[...] = kap
    z_ref[...] = lbd * _gamma(1.0 + 1.0 / kap)


_PAD2D = np.broadcast_to(
    np.asarray(N + (np.arange(EPAD - E) % (NPAD - N)), dtype=np.int32),
    (2, EPAD - E))


def kernel(x, edge_index, W1, W2, mask_rate):
    del mask_rate
    f32 = jnp.float32
    npad_extra = NPAD - N
    ei_p = jnp.concatenate(
        [edge_index, jnp.asarray(_PAD2D)], axis=1).reshape(2, NW * NCHUNK, CH)
    x_p = jnp.concatenate([x, jnp.zeros((npad_extra, x.shape[1]), f32)])
    w2_p = jnp.pad(W2, ((0, 0), (0, D2 - W2.shape[1])))

    nblk = NPAD // BLK
    r = BLK // 128
    degp = _deg_call(ei_p).reshape(2, nblk, r, 128)

    g1 = pl.pallas_call(
        _s1_body,
        grid=(nblk,),
        in_specs=[
            pl.BlockSpec((BLK, D1), lambda i: (i, 0)),
            pl.BlockSpec((D1, D1), lambda i: (0, 0)),
            pl.BlockSpec((2, 1, r, 128), lambda i: (0, i, 0, 0)),
        ],
        out_specs=pl.BlockSpec((BLK, D1), lambda i: (i, 0)),
        out_shape=jax.ShapeDtypeStruct((NPAD, D1), f32),
    )(x_p, W1, degp)

    acc1 = _gs_kernel(D1)(g1, ei_p)

    g2 = pl.pallas_call(
        _l2_body,
        grid=(nblk,),
        in_specs=[
            pl.BlockSpec((2, BLK, D1), lambda i: (0, i, 0)),
            pl.BlockSpec((BLK, D1), lambda i: (i, 0)),
            pl.BlockSpec((2, 1, r, 128), lambda i: (0, i, 0, 0)),
            pl.BlockSpec((D1, D2), lambda i: (0, 0)),
        ],
        out_specs=pl.BlockSpec((BLK, D2), lambda i: (i, 0)),
        out_shape=jax.ShapeDtypeStruct((NPAD, D2), f32),
    )(acc1, g1, degp, w2_p)

    acc2 = _gs_kernel(D2, tc_tiling=False)(g2, ei_p)

    z, lbd, kap = pl.pallas_call(
        _fin_body,
        grid=(nblk,),
        in_specs=[
            pl.BlockSpec((2, BLK, D2), lambda i: (0, i, 0)),
            pl.BlockSpec((BLK, D2), lambda i: (i, 0)),
            pl.BlockSpec((2, 1, r, 128), lambda i: (0, i, 0, 0)),
        ],
        out_specs=[
            pl.BlockSpec((BLK, 64), lambda i: (i, 0)),
            pl.BlockSpec((BLK, 64), lambda i: (i, 0)),
            pl.BlockSpec((BLK, 1), lambda i: (i, 0)),
        ],
        out_shape=[
            jax.ShapeDtypeStruct((N, 64), f32),
            jax.ShapeDtypeStruct((N, 64), f32),
            jax.ShapeDtypeStruct((N, 1), f32),
        ],
    )(acc2, g2, degp)

    return (z, lbd, kap)


# speedup vs baseline: 1.1350x; 1.0147x over previous
import functools

import numpy as np

import jax
import jax.numpy as jnp
from jax import lax
from jax.experimental import pallas as pl
from jax.experimental.pallas import tpu as pltpu
from jax.experimental.pallas import tpu_sc as plsc

N = 10000
NPAD = 10240
E = 320000
NW = 32
CH = 80
NCHUNK = E // CH // NW
STRIPE = NPAD // 16
D1 = 128
D2 = 80
BLK = 2560


@functools.lru_cache(maxsize=None)
def _gs_kernel(D, tc_tiling=True):
    mesh = plsc.VectorSubcoreMesh(core_axis_name="c", subcore_axis_name="s")

    nbuf = 4
    ring = 2 * nbuf

    @functools.partial(
        pl.kernel,
        out_type=jax.ShapeDtypeStruct((2, NPAD, D), jnp.float32),
        mesh=mesh,
        compiler_params=pltpu.CompilerParams(use_tc_tiling_on_sc=tc_tiling),
        scratch_types=(
            [pltpu.VMEM((ring, CH), jnp.int32),
             pltpu.VMEM((ring, CH), jnp.int32)] +
            [pltpu.VMEM((CH, D), jnp.float32) for _ in range(nbuf)] +
            [pltpu.VMEM_SHARED((NPAD, D), jnp.float32)] +
            [pltpu.SemaphoreType.DMA for _ in range(2 * nbuf + 2)]
        ),
    )
    def gs(g_hbm, srce_hbm, dste_hbm, out_hbm, src_v, dst_v, *bufs_and_sems):
        rows = bufs_and_sems[:nbuf]
        acc = bufs_and_sems[nbuf]
        gsem = bufs_and_sems[nbuf + 1:2 * nbuf + 1]
        ssem = bufs_and_sems[2 * nbuf + 1:3 * nbuf + 1]
        isem = bufs_and_sems[3 * nbuf + 1:3 * nbuf + 3]
        c = lax.axis_index("c")
        s = lax.axis_index("s")
        wid = s * 2 + c

        def zrow(i, _):
            for k in range(D // 16):
                rows[0][i, pl.ds(k * 16, 16)] = jnp.zeros((16,), jnp.float32)
            return _
        lax.fori_loop(0, CH, zrow, 0)
        for t in range(STRIPE // CH):
            pltpu.sync_copy(rows[0], acc.at[pl.ds(s * STRIPE + t * CH, CH)])
        plsc.subcore_barrier()

        cbase = wid * NCHUNK

        def _load_idx(j, slot, sem):
            off = (cbase + j) * CH
            pltpu.async_copy(srce_hbm.at[pl.ds(off, CH)], src_v.at[slot], sem)
            pltpu.async_copy(dste_hbm.at[pl.ds(off, CH)], dst_v.at[slot], sem)

        def _wait_idx(slot, sem):
            pltpu.make_async_copy(srce_hbm.at[pl.ds(0, CH)], src_v.at[slot], sem).wait()
            pltpu.make_async_copy(dste_hbm.at[pl.ds(0, CH)], dst_v.at[slot], sem).wait()

        for b in range(nbuf):
            _load_idx(b, b, isem[b % 2])
        for b in range(nbuf):
            _wait_idx(b, isem[b % 2])
        for b in range(nbuf):
            _load_idx(nbuf + b, nbuf + b, isem[b % 2])
        for b in range(nbuf):
            pltpu.async_copy(g_hbm.at[src_v.at[b]], rows[b], gsem[b])

        def body(jj, _):
            j0 = nbuf * jj
            for b in range(nbuf):
                j = j0 + b
                slot = lax.rem(j, ring)
                slot_n = lax.rem(j + nbuf, ring)
                pltpu.make_async_copy(g_hbm.at[src_v.at[slot]], rows[b], gsem[b]).wait()
                pltpu.async_copy(rows[b], acc.at[dst_v.at[slot]], ssem[b], add=True)
                pltpu.make_async_copy(rows[b], acc.at[dst_v.at[slot]], ssem[b]).wait()

                @pl.when(j + nbuf < NCHUNK)
                def _issue():
                    _wait_idx(slot_n, isem[b % 2])
                    pltpu.async_copy(g_hbm.at[src_v.at[slot_n]], rows[b], gsem[b])

                @pl.when(j + 2 * nbuf < NCHUNK)
                def _pre():
                    _load_idx(j + 2 * nbuf, slot, isem[b % 2])
            return _
        lax.fori_loop(0, NCHUNK // nbuf, body, 0)

        for j in range(NCHUNK - NCHUNK % nbuf, NCHUNK):
            b = j % nbuf
            slot = lax.rem(jnp.int32(j), ring)
            pltpu.make_async_copy(g_hbm.at[src_v.at[slot]], rows[b], gsem[b]).wait()
            pltpu.async_copy(rows[b], acc.at[dst_v.at[slot]], ssem[b], add=True)
            pltpu.make_async_copy(rows[b], acc.at[dst_v.at[slot]], ssem[b]).wait()

        plsc.subcore_barrier()
        for t in range(STRIPE // CH):
            r0 = s * STRIPE + t * CH
            pltpu.sync_copy(acc.at[pl.ds(r0, CH)], out_hbm.at[c, pl.ds(r0, CH)])

    return gs


def _deg_call(dst_e):
    mesh = plsc.VectorSubcoreMesh(core_axis_name="c", subcore_axis_name="s")

    @functools.partial(
        pl.kernel,
        out_type=jax.ShapeDtypeStruct((2, NPAD), jnp.float32),
        mesh=mesh,
        scratch_types=[
            pltpu.VMEM((8, CH), jnp.int32),
            pltpu.VMEM((CH,), jnp.float32),
            pltpu.VMEM((STRIPE,), jnp.float32),
            pltpu.VMEM_SHARED((NPAD,), jnp.float32),
            pltpu.SemaphoreType.DMA,
        ],
    )
    def deg(dste_hbm, out_hbm, idx_v, ones_v, zb_v, acc, isem):
        c = lax.axis_index("c")
        s = lax.axis_index("s")
        wid = s * 2 + c
        for k in range(CH // 16):
            ones_v[pl.ds(k * 16, 16)] = jnp.ones((16,), jnp.float32)
        for k in range(STRIPE // 16):
            zb_v[pl.ds(k * 16, 16)] = jnp.zeros((16,), jnp.float32)
        pltpu.sync_copy(zb_v, acc.at[pl.ds(s * STRIPE, STRIPE)])
        cbase = wid * NCHUNK
        for t in range(8):
            pltpu.async_copy(dste_hbm.at[pl.ds((cbase + t) * CH, CH)],
                             idx_v.at[t], isem)
        plsc.subcore_barrier()

        def body(j, _):
            slot = lax.rem(j, 8)
            pltpu.make_async_copy(dste_hbm.at[pl.ds(0, CH)],
                                  idx_v.at[slot], isem).wait()
            pltpu.sync_copy(ones_v, acc.at[idx_v.at[slot]], add=True)

            @pl.when(j + 8 < NCHUNK)
            def _pre():
                pltpu.async_copy(dste_hbm.at[pl.ds((cbase + j + 8) * CH, CH)],
                                 idx_v.at[slot], isem)
            return _
        lax.fori_loop(0, NCHUNK, body, 0)

        plsc.subcore_barrier()
        pltpu.sync_copy(acc.at[pl.ds(s * STRIPE, STRIPE)],
                        out_hbm.at[c, pl.ds(s * STRIPE, STRIPE)])

    return deg(dst_e)


def _softplus(x):
    return jnp.maximum(x, 0.0) + jnp.log1p(jnp.exp(-jnp.abs(x)))


def _eye128():
    return (lax.broadcasted_iota(jnp.int32, (128, 128), 0) ==
            lax.broadcasted_iota(jnp.int32, (128, 128), 1)).astype(jnp.float32)


def _to_col(mat):
    r = mat.shape[0]
    dt = lax.dot_general(_eye128(), mat, (((1,), (1,)), ((), ())),
                         preferred_element_type=jnp.float32,
                         precision=lax.Precision.HIGHEST)
    return jnp.concatenate([dt[:, k:k + 1] for k in range(r)], axis=0)


def _dinv_col(degp):
    degp = degp[:, 0]
    d = lax.rsqrt(degp[0] + degp[1] + 1.0)
    return _to_col(d)


def _s1_body(x_ref, w_ref, degp_ref, g_ref):
    dinv = _dinv_col(degp_ref[...])
    g_ref[...] = jnp.dot(x_ref[...], w_ref[...],
                         preferred_element_type=jnp.float32) * dinv


def _l2_body(acc_ref, g1_ref, degp_ref, w_ref, g2_ref):
    dinv = _dinv_col(degp_ref[...])
    a = acc_ref[...]
    out1 = _softplus((a[0] + a[1] + g1_ref[...]) * dinv)
    g2_ref[...] = jnp.dot(out1, w_ref[...],
                          preferred_element_type=jnp.float32) * dinv


def _gamma(z):
    a = jnp.float32(0.99999999999980993)
    for i, ci in enumerate([
            676.5203681218851, -1259.1392167224028, 771.32342877765313,
            -176.61502916214059, 12.507343278686905, -0.13857109526572012,
            9.9843695780195716e-6, 1.5056327351493116e-7]):
        a = a + jnp.float32(ci) / (z + jnp.float32(i))
    t = z + 6.5
    return jnp.float32(2.5066282746310002) * jnp.exp(
        (z - 0.5) * jnp.log(t) - t) * a


def _fin_body(acc_ref, g2_ref, degp_ref, z_ref, lbd_ref, kap_ref):
    dinv = _dinv_col(degp_ref[...])
    a = acc_ref[...]
    h = _softplus((a[0] + a[1] + g2_ref[...]) * dinv)
    lbd = h[:, :64]
    kap = h[:, 64:65] + 0.1
    lbd_ref[...] = lbd
    kap_ref[...] = kap
    z_ref[...] = lbd * _gamma(1.0 + 1.0 / kap)


def kernel(x, edge_index, W1, W2, mask_rate):
    del mask_rate
    f32 = jnp.float32
    src_e = edge_index[0]
    dst_e = edge_index[1]
    w2_p = jnp.pad(W2, ((0, 0), (0, D2 - W2.shape[1])))

    nblk = NPAD // BLK
    r = BLK // 128
    degp = _deg_call(dst_e).reshape(2, nblk, r, 128)

    g1 = pl.pallas_call(
        _s1_body,
        grid=(nblk,),
        in_specs=[
            pl.BlockSpec((BLK, D1), lambda i: (i, 0)),
            pl.BlockSpec((D1, D1), lambda i: (0, 0)),
            pl.BlockSpec((2, 1, r, 128), lambda i: (0, i, 0, 0)),
        ],
        out_specs=pl.BlockSpec((BLK, D1), lambda i: (i, 0)),
        out_shape=jax.ShapeDtypeStruct((N, D1), f32),
    )(x, W1, degp)

    acc1 = _gs_kernel(D1)(g1, src_e, dst_e)

    g2 = pl.pallas_call(
        _l2_body,
        grid=(nblk,),
        in_specs=[
            pl.BlockSpec((2, BLK, D1), lambda i: (0, i, 0)),
            pl.BlockSpec((BLK, D1), lambda i: (i, 0)),
            pl.BlockSpec((2, 1, r, 128), lambda i: (0, i, 0, 0)),
            pl.BlockSpec((D1, D2), lambda i: (0, 0)),
        ],
        out_specs=pl.BlockSpec((BLK, D2), lambda i: (i, 0)),
        out_shape=jax.ShapeDtypeStruct((N, D2), f32),
    )(acc1, g1, degp, w2_p)

    acc2 = _gs_kernel(D2, tc_tiling=False)(g2, src_e, dst_e)

    z, lbd, kap = pl.pallas_call(
        _fin_body,
        grid=(nblk,),
        in_specs=[
            pl.BlockSpec((2, BLK, D2), lambda i: (0, i, 0)),
            pl.BlockSpec((BLK, D2), lambda i: (i, 0)),
            pl.BlockSpec((2, 1, r, 128), lambda i: (0, i, 0, 0)),
        ],
        out_specs=[
            pl.BlockSpec((BLK, 64), lambda i: (i, 0)),
            pl.BlockSpec((BLK, 64), lambda i: (i, 0)),
            pl.BlockSpec((BLK, 1), lambda i: (i, 0)),
        ],
        out_shape=[
            jax.ShapeDtypeStruct((N, 64), f32),
            jax.ShapeDtypeStruct((N, 64), f32),
            jax.ShapeDtypeStruct((N, 1), f32),
        ],
    )(acc2, g2, degp)

    return (z, lbd, kap)
